# initial kernel scaffold (unmeasured)
import jax
import jax.numpy as jnp
from jax import lax
from jax.experimental import pallas as pl
from jax.experimental.pallas import tpu as pltpu

N_DEV = 4
EPS = 1e-5


def kernel(x, k, Wp):
    B, H, W, C = x.shape
    n_global = (2 * H) * (2 * W)

    def body(x_ref, k_ref, wp_ref, out_ref, pad_ref, stats_ref,
             send_sems, halo_sems, stats_sems):
        mx = lax.axis_index("x")
        my = lax.axis_index("y")
        my_id = mx * 2 + my
        peers = [(1 - mx, my), (mx, 1 - my), (1 - mx, 1 - my)]

        barrier_sem = pltpu.get_barrier_semaphore()
        for p in peers:
            pl.semaphore_signal(barrier_sem, inc=1, device_id=p,
                                device_id_type=pl.DeviceIdType.MESH)
        pl.semaphore_wait(barrier_sem, 3)

        xv = x_ref[...]
        s = jnp.sum(xv, axis=(1, 2))
        sq = jnp.sum(xv * xv, axis=(1, 2))
        partial = jnp.concatenate([s, sq], axis=0)
        stats_ref[pl.ds(my_id, 1), :, :] = partial[None]

        sends = []
        for i, p in enumerate(peers):
            rdma = pltpu.make_async_remote_copy(
                src_ref=stats_ref.at[pl.ds(my_id, 1)],
                dst_ref=stats_ref.at[pl.ds(my_id, 1)],
                send_sem=send_sems.at[i],
                recv_sem=stats_sems.at[pl.ds(my_id, 1)],
                device_id=p,
                device_id_type=pl.DeviceIdType.MESH,
            )
            rdma.start()
            sends.append(rdma)

        src_row = jnp.where(mx == 0, H - 1, 0)
        src_col = jnp.where(my == 0, W - 1, 0)
        dst_row = jnp.where(mx == 0, 0, H + 1)
        dst_col = jnp.where(my == 0, 0, W + 1)

        row_send = pltpu.make_async_remote_copy(
            src_ref=x_ref.at[:, pl.ds(src_row, 1), :, :],
            dst_ref=pad_ref.at[:, pl.ds(dst_row, 1), pl.ds(1, W), :],
            send_sem=send_sems.at[3],
            recv_sem=halo_sems.at[0],
            device_id=(1 - mx, my),
            device_id_type=pl.DeviceIdType.MESH,
        )
        row_send.start()
        sends.append(row_send)

        col_send = pltpu.make_async_remote_copy(
            src_ref=x_ref.at[:, :, pl.ds(src_col, 1), :],
            dst_ref=pad_ref.at[:, pl.ds(1, H), pl.ds(dst_col, 1), :],
            send_sem=send_sems.at[4],
            recv_sem=halo_sems.at[1],
            device_id=(mx, 1 - my),
            device_id_type=pl.DeviceIdType.MESH,
        )
        col_send.start()
        sends.append(col_send)

        corner_send = pltpu.make_async_remote_copy(
            src_ref=x_ref.at[:, pl.ds(src_row, 1), pl.ds(src_col, 1), :],
            dst_ref=pad_ref.at[:, pl.ds(dst_row, 1), pl.ds(dst_col, 1), :],
            send_sem=send_sems.at[5],
            recv_sem=halo_sems.at[2],
            device_id=(1 - mx, 1 - my),
            device_id_type=pl.DeviceIdType.MESH,
        )
        corner_send.start()
        sends.append(corner_send)

        pad_ref[:, pl.ds(1, H), pl.ds(1, W), :] = xv

        rcv_row = jnp.where(mx == 0, H + 1, 0)
        rcv_col = jnp.where(my == 0, W + 1, 0)
        row_recv = pltpu.make_async_remote_copy(
            src_ref=x_ref.at[:, pl.ds(src_row, 1), :, :],
            dst_ref=pad_ref.at[:, pl.ds(rcv_row, 1), pl.ds(1, W), :],
            send_sem=send_sems.at[3],
            recv_sem=halo_sems.at[0],
            device_id=(1 - mx, my),
            device_id_type=pl.DeviceIdType.MESH,
        )
        row_recv.wait_recv()
        col_recv = pltpu.make_async_remote_copy(
            src_ref=x_ref.at[:, :, pl.ds(src_col, 1), :],
            dst_ref=pad_ref.at[:, pl.ds(1, H), pl.ds(rcv_col, 1), :],
            send_sem=send_sems.at[4],
            recv_sem=halo_sems.at[1],
            device_id=(mx, 1 - my),
            device_id_type=pl.DeviceIdType.MESH,
        )
        col_recv.wait_recv()
        corner_recv = pltpu.make_async_remote_copy(
            src_ref=x_ref.at[:, pl.ds(src_row, 1), pl.ds(src_col, 1), :],
            dst_ref=pad_ref.at[:, pl.ds(rcv_row, 1), pl.ds(rcv_col, 1), :],
            send_sem=send_sems.at[5],
            recv_sem=halo_sems.at[2],
            device_id=(1 - mx, 1 - my),
            device_id_type=pl.DeviceIdType.MESH,
        )
        corner_recv.wait_recv()

        @pl.when(my == 0)
        def _():
            pad_ref[:, :, pl.ds(0, 1), :] = pad_ref[:, :, pl.ds(1, 1), :]

        @pl.when(my == 1)
        def _():
            pad_ref[:, :, pl.ds(W + 1, 1), :] = pad_ref[:, :, pl.ds(W, 1), :]

        @pl.when(mx == 0)
        def _():
            pad_ref[:, pl.ds(0, 1), :, :] = pad_ref[:, pl.ds(1, 1), :, :]

        @pl.when(mx == 1)
        def _():
            pad_ref[:, pl.ds(H + 1, 1), :, :] = pad_ref[:, pl.ds(H, 1), :, :]

        for p in peers:
            pid = p[0] * 2 + p[1]
            stat_recv = pltpu.make_async_remote_copy(
                src_ref=stats_ref.at[pl.ds(my_id, 1)],
                dst_ref=stats_ref.at[pl.ds(pid, 1)],
                send_sem=send_sems.at[0],
                recv_sem=stats_sems.at[pl.ds(pid, 1)],
                device_id=p,
                device_id_type=pl.DeviceIdType.MESH,
            )
            stat_recv.wait_recv()

        tot = jnp.sum(stats_ref[...], axis=0)
        mean = tot[0:B] / n_global
        ex2 = tot[B:2 * B] / n_global
        var = ex2 - mean * mean
        rstd = lax.rsqrt(var + EPS)
        mean_b = mean[:, None, None, :]
        rstd_b = rstd[:, None, None, :]

        kv = k_ref[...]
        conv = jnp.zeros((B, H, W, C), jnp.float32)
        for di in range(3):
            for dj in range(3):
                tap = pad_ref[:, pl.ds(di, H), pl.ds(dj, W), :]
                hp = (tap - mean_b) * rstd_b
                conv = conv + hp * kv[di, dj][None, None, None, :]

        a = conv * (1.0 / (1.0 + jnp.exp(-conv)))
        y = jnp.dot(a.reshape(B * H * W, C), wp_ref[...],
                    preferred_element_type=jnp.float32)
        out_ref[...] = xv + y.reshape(B, H, W, C)

        for rdma in sends:
            rdma.wait_send()

    return pl.pallas_call(
        body,
        out_shape=jax.ShapeDtypeStruct((B, H, W, C), jnp.float32),
        in_specs=[
            pl.BlockSpec(memory_space=pltpu.VMEM),
            pl.BlockSpec(memory_space=pltpu.VMEM),
            pl.BlockSpec(memory_space=pltpu.VMEM),
        ],
        out_specs=pl.BlockSpec(memory_space=pltpu.VMEM),
        scratch_shapes=[
            pltpu.VMEM((B, H + 2, W + 2, C), jnp.float32),
            pltpu.VMEM((N_DEV, 2 * B, C), jnp.float32),
            pltpu.SemaphoreType.DMA((6,)),
            pltpu.SemaphoreType.DMA((3,)),
            pltpu.SemaphoreType.DMA((4,)),
        ],
        compiler_params=pltpu.CompilerParams(collective_id=0),
    )(x, k, Wp)


# baseline (device time: 18410 ns/iter reference)
import jax
import jax.numpy as jnp
from jax import lax
from jax.experimental import pallas as pl
from jax.experimental.pallas import tpu as pltpu

N_DEV = 4
EPS = 1e-5


def kernel(x, k, Wp):
    B, H, W, C = x.shape
    n_global = (2 * H) * (2 * W)

    def body(x_ref, k_ref, wp_ref, out_ref, pad_ref, stats_ref,
             send_sems, halo_sems, stats_sems):
        mx = lax.axis_index("x")
        my = lax.axis_index("y")
        my_id = mx * 2 + my
        peers = [(1 - mx, my), (mx, 1 - my), (1 - mx, 1 - my)]

        barrier_sem = pltpu.get_barrier_semaphore()
        for p in peers:
            pl.semaphore_signal(barrier_sem, inc=1, device_id=p,
                                device_id_type=pl.DeviceIdType.MESH)
        pl.semaphore_wait(barrier_sem, 3)

        xv = x_ref[...]
        s = jnp.sum(xv, axis=(1, 2))
        sq = jnp.sum(xv * xv, axis=(1, 2))
        partial = jnp.concatenate([s, sq], axis=0)
        stats_ref[pl.ds(my_id, 1), :, :] = partial[None]

        sends = []
        for i, p in enumerate(peers):
            rdma = pltpu.make_async_remote_copy(
                src_ref=stats_ref.at[pl.ds(my_id, 1)],
                dst_ref=stats_ref.at[pl.ds(my_id, 1)],
                send_sem=send_sems.at[i],
                recv_sem=stats_sems.at[my_id],
                device_id=p,
                device_id_type=pl.DeviceIdType.MESH,
            )
            rdma.start()
            sends.append(rdma)

        src_row = jnp.where(mx == 0, H - 1, 0)
        src_col = jnp.where(my == 0, W - 1, 0)
        dst_row = jnp.where(mx == 0, 0, H + 1)
        dst_col = jnp.where(my == 0, 0, W + 1)

        row_send = pltpu.make_async_remote_copy(
            src_ref=x_ref.at[:, pl.ds(src_row, 1), :, :],
            dst_ref=pad_ref.at[:, pl.ds(dst_row, 1), pl.ds(1, W), :],
            send_sem=send_sems.at[3],
            recv_sem=halo_sems.at[0],
            device_id=(1 - mx, my),
            device_id_type=pl.DeviceIdType.MESH,
        )
        row_send.start()
        sends.append(row_send)

        col_send = pltpu.make_async_remote_copy(
            src_ref=x_ref.at[:, :, pl.ds(src_col, 1), :],
            dst_ref=pad_ref.at[:, pl.ds(1, H), pl.ds(dst_col, 1), :],
            send_sem=send_sems.at[4],
            recv_sem=halo_sems.at[1],
            device_id=(mx, 1 - my),
            device_id_type=pl.DeviceIdType.MESH,
        )
        col_send.start()
        sends.append(col_send)

        corner_send = pltpu.make_async_remote_copy(
            src_ref=x_ref.at[:, pl.ds(src_row, 1), pl.ds(src_col, 1), :],
            dst_ref=pad_ref.at[:, pl.ds(dst_row, 1), pl.ds(dst_col, 1), :],
            send_sem=send_sems.at[5],
            recv_sem=halo_sems.at[2],
            device_id=(1 - mx, 1 - my),
            device_id_type=pl.DeviceIdType.MESH,
        )
        corner_send.start()
        sends.append(corner_send)

        pad_ref[:, pl.ds(1, H), pl.ds(1, W), :] = xv

        rcv_row = jnp.where(mx == 0, H + 1, 0)
        rcv_col = jnp.where(my == 0, W + 1, 0)
        row_recv = pltpu.make_async_remote_copy(
            src_ref=x_ref.at[:, pl.ds(src_row, 1), :, :],
            dst_ref=pad_ref.at[:, pl.ds(rcv_row, 1), pl.ds(1, W), :],
            send_sem=send_sems.at[3],
            recv_sem=halo_sems.at[0],
            device_id=(1 - mx, my),
            device_id_type=pl.DeviceIdType.MESH,
        )
        row_recv.wait_recv()
        col_recv = pltpu.make_async_remote_copy(
            src_ref=x_ref.at[:, :, pl.ds(src_col, 1), :],
            dst_ref=pad_ref.at[:, pl.ds(1, H), pl.ds(rcv_col, 1), :],
            send_sem=send_sems.at[4],
            recv_sem=halo_sems.at[1],
            device_id=(mx, 1 - my),
            device_id_type=pl.DeviceIdType.MESH,
        )
        col_recv.wait_recv()
        corner_recv = pltpu.make_async_remote_copy(
            src_ref=x_ref.at[:, pl.ds(src_row, 1), pl.ds(src_col, 1), :],
            dst_ref=pad_ref.at[:, pl.ds(rcv_row, 1), pl.ds(rcv_col, 1), :],
            send_sem=send_sems.at[5],
            recv_sem=halo_sems.at[2],
            device_id=(1 - mx, 1 - my),
            device_id_type=pl.DeviceIdType.MESH,
        )
        corner_recv.wait_recv()

        @pl.when(my == 0)
        def _():
            pad_ref[:, :, pl.ds(0, 1), :] = pad_ref[:, :, pl.ds(1, 1), :]

        @pl.when(my == 1)
        def _():
            pad_ref[:, :, pl.ds(W + 1, 1), :] = pad_ref[:, :, pl.ds(W, 1), :]

        @pl.when(mx == 0)
        def _():
            pad_ref[:, pl.ds(0, 1), :, :] = pad_ref[:, pl.ds(1, 1), :, :]

        @pl.when(mx == 1)
        def _():
            pad_ref[:, pl.ds(H + 1, 1), :, :] = pad_ref[:, pl.ds(H, 1), :, :]

        for p in peers:
            pid = p[0] * 2 + p[1]
            stat_recv = pltpu.make_async_remote_copy(
                src_ref=stats_ref.at[pl.ds(my_id, 1)],
                dst_ref=stats_ref.at[pl.ds(pid, 1)],
                send_sem=send_sems.at[0],
                recv_sem=stats_sems.at[pid],
                device_id=p,
                device_id_type=pl.DeviceIdType.MESH,
            )
            stat_recv.wait_recv()

        tot = jnp.sum(stats_ref[...], axis=0)
        mean = tot[0:B] / n_global
        ex2 = tot[B:2 * B] / n_global
        var = ex2 - mean * mean
        rstd = lax.rsqrt(var + EPS)
        mean_b = mean[:, None, None, :]
        rstd_b = rstd[:, None, None, :]

        kv = k_ref[...]
        conv = jnp.zeros((B, H, W, C), jnp.float32)
        for di in range(3):
            for dj in range(3):
                tap = pad_ref[:, pl.ds(di, H), pl.ds(dj, W), :]
                hp = (tap - mean_b) * rstd_b
                conv = conv + hp * kv[di, dj][None, None, None, :]

        a = conv * (1.0 / (1.0 + jnp.exp(-conv)))
        y = jnp.dot(a.reshape(B * H * W, C), wp_ref[...],
                    preferred_element_type=jnp.float32)
        out_ref[...] = xv + y.reshape(B, H, W, C)

        for rdma in sends:
            rdma.wait_send()

    return pl.pallas_call(
        body,
        out_shape=jax.ShapeDtypeStruct((B, H, W, C), jnp.float32),
        in_specs=[
            pl.BlockSpec(memory_space=pltpu.VMEM),
            pl.BlockSpec(memory_space=pltpu.VMEM),
            pl.BlockSpec(memory_space=pltpu.VMEM),
        ],
        out_specs=pl.BlockSpec(memory_space=pltpu.VMEM),
        scratch_shapes=[
            pltpu.VMEM((B, H + 2, W + 2, C), jnp.float32),
            pltpu.VMEM((N_DEV, 2 * B, C), jnp.float32),
            pltpu.SemaphoreType.DMA((6,)),
            pltpu.SemaphoreType.DMA((3,)),
            pltpu.SemaphoreType.DMA((4,)),
        ],
        compiler_params=pltpu.CompilerParams(collective_id=0),
    )(x, k, Wp)


# device time: 16253 ns/iter; 1.1327x vs baseline; 1.1327x over previous
import jax
import jax.numpy as jnp
from jax import lax
from jax.experimental import pallas as pl
from jax.experimental.pallas import tpu as pltpu

N_DEV = 4
EPS = 1e-5


def kernel(x, k, Wp):
    B, H, W, C = x.shape
    n_global = (2 * H) * (2 * W)

    def body(x_ref, k_ref, wp_ref, out_ref, pad_ref, stats_ref,
             send_sems, halo_sems, stats_sems):
        mx = lax.axis_index("x")
        my = lax.axis_index("y")
        my_id = mx * 2 + my
        peers = [(1 - mx, my), (mx, 1 - my), (1 - mx, 1 - my)]

        barrier_sem = pltpu.get_barrier_semaphore()
        for p in peers:
            pl.semaphore_signal(barrier_sem, inc=1, device_id=p,
                                device_id_type=pl.DeviceIdType.MESH)
        pl.semaphore_wait(barrier_sem, 3)

        xv = x_ref[...]
        s = jnp.sum(xv, axis=(1, 2))
        sq = jnp.sum(xv * xv, axis=(1, 2))
        partial = jnp.concatenate([s, sq], axis=0)
        stats_ref[pl.ds(my_id, 1), :, :] = partial[None]

        sends = []
        for i, p in enumerate(peers):
            rdma = pltpu.make_async_remote_copy(
                src_ref=stats_ref.at[pl.ds(my_id, 1)],
                dst_ref=stats_ref.at[pl.ds(my_id, 1)],
                send_sem=send_sems.at[i],
                recv_sem=stats_sems.at[my_id],
                device_id=p,
                device_id_type=pl.DeviceIdType.MESH,
            )
            rdma.start()
            sends.append(rdma)

        src_row = jnp.where(mx == 0, H - 1, 0)
        src_col = jnp.where(my == 0, W - 1, 0)
        dst_row = jnp.where(mx == 0, 0, H + 1)
        dst_col = jnp.where(my == 0, 0, W + 1)

        row_send = pltpu.make_async_remote_copy(
            src_ref=x_ref.at[:, pl.ds(src_row, 1), :, :],
            dst_ref=pad_ref.at[:, pl.ds(dst_row, 1), pl.ds(1, W), :],
            send_sem=send_sems.at[3],
            recv_sem=halo_sems.at[0],
            device_id=(1 - mx, my),
            device_id_type=pl.DeviceIdType.MESH,
        )
        row_send.start()
        sends.append(row_send)

        col_send = pltpu.make_async_remote_copy(
            src_ref=x_ref.at[:, :, pl.ds(src_col, 1), :],
            dst_ref=pad_ref.at[:, pl.ds(1, H), pl.ds(dst_col, 1), :],
            send_sem=send_sems.at[4],
            recv_sem=halo_sems.at[1],
            device_id=(mx, 1 - my),
            device_id_type=pl.DeviceIdType.MESH,
        )
        col_send.start()
        sends.append(col_send)

        corner_send = pltpu.make_async_remote_copy(
            src_ref=x_ref.at[:, pl.ds(src_row, 1), pl.ds(src_col, 1), :],
            dst_ref=pad_ref.at[:, pl.ds(dst_row, 1), pl.ds(dst_col, 1), :],
            send_sem=send_sems.at[5],
            recv_sem=halo_sems.at[2],
            device_id=(1 - mx, 1 - my),
            device_id_type=pl.DeviceIdType.MESH,
        )
        corner_send.start()
        sends.append(corner_send)

        pad_ref[:, pl.ds(1, H), pl.ds(1, W), :] = xv

        rcv_row = jnp.where(mx == 0, H + 1, 0)
        rcv_col = jnp.where(my == 0, W + 1, 0)
        row_recv = pltpu.make_async_remote_copy(
            src_ref=x_ref.at[:, pl.ds(src_row, 1), :, :],
            dst_ref=pad_ref.at[:, pl.ds(rcv_row, 1), pl.ds(1, W), :],
            send_sem=send_sems.at[3],
            recv_sem=halo_sems.at[0],
            device_id=(1 - mx, my),
            device_id_type=pl.DeviceIdType.MESH,
        )
        row_recv.wait_recv()
        col_recv = pltpu.make_async_remote_copy(
            src_ref=x_ref.at[:, :, pl.ds(src_col, 1), :],
            dst_ref=pad_ref.at[:, pl.ds(1, H), pl.ds(rcv_col, 1), :],
            send_sem=send_sems.at[4],
            recv_sem=halo_sems.at[1],
            device_id=(mx, 1 - my),
            device_id_type=pl.DeviceIdType.MESH,
        )
        col_recv.wait_recv()
        corner_recv = pltpu.make_async_remote_copy(
            src_ref=x_ref.at[:, pl.ds(src_row, 1), pl.ds(src_col, 1), :],
            dst_ref=pad_ref.at[:, pl.ds(rcv_row, 1), pl.ds(rcv_col, 1), :],
            send_sem=send_sems.at[5],
            recv_sem=halo_sems.at[2],
            device_id=(1 - mx, 1 - my),
            device_id_type=pl.DeviceIdType.MESH,
        )
        corner_recv.wait_recv()

        @pl.when(my == 0)
        def _():
            pad_ref[:, :, pl.ds(0, 1), :] = pad_ref[:, :, pl.ds(1, 1), :]

        @pl.when(my == 1)
        def _():
            pad_ref[:, :, pl.ds(W + 1, 1), :] = pad_ref[:, :, pl.ds(W, 1), :]

        @pl.when(mx == 0)
        def _():
            pad_ref[:, pl.ds(0, 1), :, :] = pad_ref[:, pl.ds(1, 1), :, :]

        @pl.when(mx == 1)
        def _():
            pad_ref[:, pl.ds(H + 1, 1), :, :] = pad_ref[:, pl.ds(H, 1), :, :]

        for p in peers:
            pid = p[0] * 2 + p[1]
            stat_recv = pltpu.make_async_remote_copy(
                src_ref=stats_ref.at[pl.ds(my_id, 1)],
                dst_ref=stats_ref.at[pl.ds(pid, 1)],
                send_sem=send_sems.at[0],
                recv_sem=stats_sems.at[pid],
                device_id=p,
                device_id_type=pl.DeviceIdType.MESH,
            )
            stat_recv.wait_recv()

        tot = jnp.sum(stats_ref[...], axis=0)
        mean = tot[0:B] / n_global
        ex2 = tot[B:2 * B] / n_global
        var = ex2 - mean * mean
        rstd = lax.rsqrt(var + EPS)
        mean_b = mean[:, None, None, :]
        rstd_b = rstd[:, None, None, :]

        kv = k_ref[...]
        conv_raw = jnp.zeros((B, H, W, C), jnp.float32)
        for di in range(3):
            for dj in range(3):
                tap = pad_ref[:, pl.ds(di, H), pl.ds(dj, W), :]
                conv_raw = conv_raw + tap * kv[di, dj][None, None, None, :]
        ksum = jnp.sum(kv, axis=(0, 1))[None, None, None, :]
        conv = (conv_raw - mean_b * ksum) * rstd_b

        a = conv * (1.0 / (1.0 + jnp.exp(-conv)))
        y = jnp.dot(a.astype(jnp.bfloat16).reshape(B * H * W, C),
                    wp_ref[...].astype(jnp.bfloat16),
                    preferred_element_type=jnp.float32)
        out_ref[...] = xv + y.reshape(B, H, W, C)

        for rdma in sends:
            rdma.wait_send()

    return pl.pallas_call(
        body,
        out_shape=jax.ShapeDtypeStruct((B, H, W, C), jnp.float32),
        in_specs=[
            pl.BlockSpec(memory_space=pltpu.VMEM),
            pl.BlockSpec(memory_space=pltpu.VMEM),
            pl.BlockSpec(memory_space=pltpu.VMEM),
        ],
        out_specs=pl.BlockSpec(memory_space=pltpu.VMEM),
        scratch_shapes=[
            pltpu.VMEM((B, H + 2, W + 2, C), jnp.float32),
            pltpu.VMEM((N_DEV, 2 * B, C), jnp.float32),
            pltpu.SemaphoreType.DMA((6,)),
            pltpu.SemaphoreType.DMA((3,)),
            pltpu.SemaphoreType.DMA((4,)),
        ],
        compiler_params=pltpu.CompilerParams(collective_id=0),
    )(x, k, Wp)
